# baseline (speedup 1.0000x reference)
import functools

import jax
import jax.numpy as jnp
from jax.experimental import pallas as pl
from jax.experimental.pallas import tpu as pltpu


def _leaky(v):
    return jnp.where(v > 0, v, 0.2 * v)


def _gatv2(h, src, dst, ea, emask, Wl, bl, Wr, br, We, att, bias, N):
    xl = h @ Wl + bl
    xr = h @ Wr + br
    mcnt = jnp.maximum(emask.sum().astype(jnp.float32), 1.0)
    mean_ea = (ea * emask[:, None].astype(ea.dtype)).sum(0) / mcnt
    loop = jnp.arange(N, dtype=src.dtype)
    src_f = jnp.concatenate([src, loop])
    dst_f = jnp.concatenate([dst, loop])
    e_f = jnp.concatenate([ea @ We, jnp.broadcast_to(mean_ea @ We, (N, Wl.shape[1]))], axis=0)
    m = _leaky(xl[src_f] + xr[dst_f] + e_f)
    logits = (m * att).sum(-1)
    fmask = jnp.concatenate([emask, jnp.ones((N,), dtype=bool)])
    logits = jnp.where(fmask, logits, -1e30)
    lmax = jax.lax.stop_gradient(jax.ops.segment_max(logits, dst_f, num_segments=N))
    ex = jnp.exp(logits - lmax[dst_f]) * fmask.astype(logits.dtype)
    den = jax.ops.segment_sum(ex, dst_f, num_segments=N)
    alpha = ex / (den[dst_f] + 1e-16)
    out = jax.ops.segment_sum(xl[src_f] * alpha[:, None], dst_f, num_segments=N)
    return out + bias


def _final_matmul_kernel(p_ref, w_ref, b_ref, o_ref):
    o_ref[...] = jnp.dot(p_ref[...], w_ref[...],
                         preferred_element_type=jnp.float32) + b_ref[...]


def _final_matmul(pooled, Wf, bf):
    G, OD = pooled.shape[0], Wf.shape[1]
    return pl.pallas_call(
        _final_matmul_kernel,
        out_shape=jax.ShapeDtypeStruct((G, OD), jnp.float32),
    )(pooled, Wf, bf[None, :])


def kernel(x, edge_attr, emb, Wn, bn, Wed, bed, W1l, b1l, W1r, b1r, W1e, att1, bias1,
           W2l, b2l, W2r, b2r, W2e, att2, bias2, Wf, bf, edge_index, batch):
    N = x.shape[0]
    G = 64
    ea2 = jnp.concatenate([edge_attr, edge_attr], axis=0)
    src, dst = edge_index[0], edge_index[1]
    emask = src != dst
    idx = jnp.argmax(x[:, :-1], axis=1)
    h = jnp.concatenate([emb[idx], x[:, -1:]], axis=1)
    h = h @ Wn + bn
    ea = ea2 @ Wed + bed
    h = jax.nn.relu(_gatv2(h, src, dst, ea, emask, W1l, b1l, W1r, b1r, W1e, att1, bias1, N))
    h = jax.nn.relu(_gatv2(h, src, dst, ea, emask, W2l, b2l, W2r, b2r, W2e, att2, bias2, N))
    sums = jax.ops.segment_sum(h, batch, num_segments=G)
    cnt = jax.ops.segment_sum(jnp.ones((N,), jnp.float32), batch, num_segments=G)
    pooled = sums / jnp.maximum(cnt, 1.0)[:, None]
    return _final_matmul(pooled, Wf, bf)


# speedup vs baseline: 11.5364x; 11.5364x over previous
import functools

import jax
import jax.numpy as jnp
from jax import lax
from jax.experimental import pallas as pl
from jax.experimental.pallas import tpu as pltpu
from jax.experimental.pallas import tpu_sc as plsc

F32 = jnp.float32


def _leaky(v):
    return jnp.where(v > 0, v, 0.2 * v)


def _eared_body(ea_ref, m0_ref, m1_ref, o_ref):
    j = pl.program_id(0)

    @pl.when(j == 0)
    def _():
        o_ref[...] = jnp.zeros_like(o_ref)

    w = m0_ref[0, 0] + m1_ref[0, 0]
    s4 = (ea_ref[...] * w[:, None]).sum(0)
    cnt = w.sum()
    o_ref[...] += jnp.concatenate([s4, cnt[None], jnp.zeros((3,), F32)])[None, :]


def _ea_reduce(ea, m0, m1, NJ, Bj):
    return pl.pallas_call(
        _eared_body,
        grid=(NJ,),
        in_specs=[
            pl.BlockSpec((Bj, 4), lambda j: (j, 0)),
            pl.BlockSpec((1, 1, Bj), lambda j: (j, 0, 0)),
            pl.BlockSpec((1, 1, Bj), lambda j: (j, 0, 0)),
        ],
        out_specs=pl.BlockSpec((1, 8), lambda j: (0, 0)),
        out_shape=jax.ShapeDtypeStruct((1, 8), F32),
    )(ea, m0, m1)


def _emit_tab(h, Wl, bl, Wr, br, att, efm):
    B = h.shape[0]
    xl = jnp.dot(h, Wl, preferred_element_type=F32) + bl
    xr = jnp.dot(h, Wr, preferred_element_type=F32) + br
    selfl = (_leaky(xl + xr + efm) * att).sum(1)
    return jnp.concatenate(
        [h, jnp.ones((B, 1), F32), selfl[:, None], jnp.zeros((B, 6), F32)], axis=1)


def _front_body(x_ref, T_ref, wnl_ref, bn_ref, W1l_ref, b1l_ref, W1r_ref,
                b1r_ref, att1_ref, efm1_ref, o_ref):
    xa = x_ref[:, :118]
    B = xa.shape[0]
    mx = xa.max(axis=1, keepdims=True)
    ii = lax.broadcasted_iota(jnp.int32, (B, 118), 1)
    amin = jnp.min(jnp.where(xa == mx, ii, jnp.int32(1 << 30)), axis=1)
    onehot = (ii == amin[:, None]).astype(F32)
    h = (jnp.dot(onehot, T_ref[...], preferred_element_type=F32)
         + x_ref[:, 118:119] * wnl_ref[...] + bn_ref[...])
    o_ref[...] = _emit_tab(h, W1l_ref[...], b1l_ref[...], W1r_ref[...],
                           b1r_ref[...], att1_ref[...], efm1_ref[...])


def _front(x, T, wnl, bn, W1l, b1l, W1r, b1r, att1, efm1, N, NB, Bn):
    full = lambda a: pl.BlockSpec(a.shape, lambda i: (0,) * a.ndim)
    return pl.pallas_call(
        _front_body,
        grid=(NB,),
        in_specs=[pl.BlockSpec((Bn, 119), lambda i: (i, 0))]
        + [full(a) for a in (T, wnl, bn, W1l, b1l, W1r, b1r, att1, efm1)],
        out_specs=pl.BlockSpec((Bn, 32), lambda i: (i, 0)),
        out_shape=jax.ShapeDtypeStruct((N, 32), F32),
    )(x, T, wnl, bn, W1l, b1l, W1r, b1r, att1, efm1)


def _edge_body(gl_ref, gr_ref, ea_ref, m0_ref, m1_ref, Wl_ref, Wr_ref,
               Wf4_ref, bsum_ref, att_ref, o_ref):
    two, Bj, _ = gl_ref.shape
    gl = gl_ref[...].reshape(2 * Bj, 32)
    gr = gr_ref[...].reshape(2 * Bj, 32)
    ef = jnp.dot(ea_ref[...], Wf4_ref[...], preferred_element_type=F32)
    z = (jnp.dot(gl[:, :24], Wl_ref[...], preferred_element_type=F32)
         + jnp.dot(gr[:, :24], Wr_ref[...], preferred_element_type=F32)
         + jnp.concatenate([ef, ef], axis=0) + bsum_ref[...])
    lg = (_leaky(z) * att_ref[...]).sum(1) - gr[:, 25]
    wm = jnp.concatenate([m0_ref[0, 0], m1_ref[0, 0]])
    ex = jnp.exp(jnp.minimum(lg, 80.0)) * wm
    o_ref[...] = (ex[:, None] * gl).reshape(2, Bj, 32)


def _edge_pass(gl, gr, ea, m0, m1, Wl, Wr, Wf4, bsum, att, E, NJ, Bj):
    full = lambda a: pl.BlockSpec(a.shape, lambda j: (0,) * a.ndim)
    return pl.pallas_call(
        _edge_body,
        grid=(NJ,),
        in_specs=[
            pl.BlockSpec((2, Bj, 32), lambda j: (0, j, 0)),
            pl.BlockSpec((2, Bj, 32), lambda j: (0, j, 0)),
            pl.BlockSpec((Bj, 4), lambda j: (j, 0)),
            pl.BlockSpec((1, 1, Bj), lambda j: (j, 0, 0)),
            pl.BlockSpec((1, 1, Bj), lambda j: (j, 0, 0)),
        ] + [full(a) for a in (Wl, Wr, Wf4, bsum, att)],
        out_specs=pl.BlockSpec((2, Bj, 32), lambda j: (0, j, 0)),
        out_shape=jax.ShapeDtypeStruct((2, E // 2, 32), F32),
    )(gl, gr, ea, m0, m1, Wl, Wr, Wf4, bsum, att)


def _comb_tab_body(acc_ref, tab_ref, Wl_ref, bb_ref, W2l_ref, b2l_ref,
                   W2r_ref, b2r_ref, att2_ref, efm2_ref, o_ref):
    a = acc_ref[0] + acc_ref[1]
    h = tab_ref[:, :24]
    num = a[:, :24] + h
    den = a[:, 24:25] + 1.0
    hn = jax.nn.relu(
        jnp.dot(num / den, Wl_ref[...], preferred_element_type=F32) + bb_ref[...])
    o_ref[...] = _emit_tab(hn, W2l_ref[...], b2l_ref[...], W2r_ref[...],
                           b2r_ref[...], att2_ref[...], efm2_ref[...])


def _comb_tab(acc, tab, Wl, bb, W2l, b2l, W2r, b2r, att2, efm2, N, NB, Bn):
    full = lambda a: pl.BlockSpec(a.shape, lambda i: (0,) * a.ndim)
    return pl.pallas_call(
        _comb_tab_body,
        grid=(NB,),
        in_specs=[
            pl.BlockSpec((2, Bn, 32), lambda i: (0, i, 0)),
            pl.BlockSpec((Bn, 32), lambda i: (i, 0)),
        ] + [full(a) for a in (Wl, bb, W2l, b2l, W2r, b2r, att2, efm2)],
        out_specs=pl.BlockSpec((Bn, 32), lambda i: (i, 0)),
        out_shape=jax.ShapeDtypeStruct((N, 32), F32),
    )(acc, tab, Wl, bb, W2l, b2l, W2r, b2r, att2, efm2)


def _comb_h_body(acc_ref, tab_ref, Wl_ref, bb_ref, o_ref):
    a = acc_ref[0] + acc_ref[1]
    h = tab_ref[:, :24]
    num = a[:, :24] + h
    den = a[:, 24:25] + 1.0
    o_ref[...] = jax.nn.relu(
        jnp.dot(num / den, Wl_ref[...], preferred_element_type=F32) + bb_ref[...])


def _comb_h(acc, tab, Wl, bb, N, NB, Bn):
    full = lambda a: pl.BlockSpec(a.shape, lambda i: (0,) * a.ndim)
    F = Wl.shape[1]
    return pl.pallas_call(
        _comb_h_body,
        grid=(NB,),
        in_specs=[
            pl.BlockSpec((2, Bn, 32), lambda i: (0, i, 0)),
            pl.BlockSpec((Bn, 32), lambda i: (i, 0)),
        ] + [full(a) for a in (Wl, bb)],
        out_specs=pl.BlockSpec((Bn, F), lambda i: (i, 0)),
        out_shape=jax.ShapeDtypeStruct((N, F), F32),
    )(acc, tab, Wl, bb)


def _pool_body(h_ref, b_ref, Wf_ref, bf_ref, o_ref, s_ref, c_ref):
    i = pl.program_id(0)
    nb = pl.num_programs(0)

    @pl.when(i == 0)
    def _():
        s_ref[...] = jnp.zeros_like(s_ref)
        c_ref[...] = jnp.zeros_like(c_ref)

    b = b_ref[0, 0, :]
    Bn = b.shape[0]
    onehot = (b[:, None] == lax.broadcasted_iota(jnp.int32, (Bn, 64), 1)).astype(F32)
    s_ref[...] += lax.dot_general(onehot, h_ref[...], (((0,), (0,)), ((), ())),
                                  preferred_element_type=F32)
    c_ref[...] += onehot.sum(0)[None, :]

    @pl.when(i == nb - 1)
    def _():
        pooled = s_ref[...] / jnp.maximum(c_ref[0], 1.0)[:, None]
        o_ref[...] = (jnp.dot(pooled, Wf_ref[...], preferred_element_type=F32)
                      + bf_ref[...])


def _pool_final(h, batch3, Wf, bf, NB, Bn):
    full = lambda a: pl.BlockSpec(a.shape, lambda i: (0,) * a.ndim)
    F = h.shape[1]
    return pl.pallas_call(
        _pool_body,
        grid=(NB,),
        in_specs=[
            pl.BlockSpec((Bn, F), lambda i: (i, 0)),
            pl.BlockSpec((1, 1, Bn), lambda i: (i, 0, 0)),
        ] + [full(a) for a in (Wf, bf)],
        out_specs=pl.BlockSpec((64, 64), lambda i: (0, 0)),
        out_shape=jax.ShapeDtypeStruct((64, 64), F32),
        scratch_shapes=[pltpu.VMEM((64, F), F32), pltpu.VMEM((1, 64), F32)],
    )(h, batch3, Wf, bf)


IDXW = 125
KJ = 8


def _make_gather(N, E):
    ROWS = E // IDXW
    ROWS_W = ROWS // 32
    CH = ROWS_W // KJ
    EW = ROWS_W * IDXW
    mesh = plsc.VectorSubcoreMesh(core_axis_name="c", subcore_axis_name="s")

    @functools.partial(
        pl.kernel,
        out_type=[jax.ShapeDtypeStruct((E, 32), F32),
                  jax.ShapeDtypeStruct((E, 32), F32)],
        mesh=mesh,
        compiler_params=pltpu.CompilerParams(use_tc_tiling_on_sc=False),
        scratch_types=[
            pltpu.VMEM((KJ, IDXW), jnp.int32),
            pltpu.VMEM((KJ, IDXW), jnp.int32),
            pltpu.VMEM((KJ * IDXW, 32), F32),
            pltpu.VMEM((KJ * IDXW, 32), F32),
            pltpu.SemaphoreType.DMA,
            pltpu.SemaphoreType.DMA,
        ],
    )
    def gather_k(tab_hbm, src2_hbm, dst2_hbm, gl_hbm, gr_hbm,
                 idxs_v, idxd_v, rs_v, rd_v, sems, semd):
        wid = lax.axis_index("s") * 2 + lax.axis_index("c")
        row0 = wid * ROWS_W
        e0 = wid * EW

        def body(i, carry):
            r0 = row0 + i * KJ
            pltpu.sync_copy(src2_hbm.at[pl.ds(r0, KJ)], idxs_v)
            pltpu.sync_copy(dst2_hbm.at[pl.ds(r0, KJ)], idxd_v)
            hs = []
            for j in range(KJ):
                hs.append(pltpu.async_copy(
                    tab_hbm.at[idxs_v.at[j]],
                    rs_v.at[pl.ds(j * IDXW, IDXW)], sems))
                hs.append(pltpu.async_copy(
                    tab_hbm.at[idxd_v.at[j]],
                    rd_v.at[pl.ds(j * IDXW, IDXW)], semd))
            for h in hs:
                h.wait()
            base = e0 + i * (KJ * IDXW)
            pltpu.sync_copy(rs_v, gl_hbm.at[pl.ds(base, KJ * IDXW)])
            pltpu.sync_copy(rd_v, gr_hbm.at[pl.ds(base, KJ * IDXW)])
            return carry

        lax.fori_loop(0, CH, body, 0)

    return gather_k


def _make_scatter(N, E):
    KJS = 4
    ROWS = E // IDXW
    ROWS_W = ROWS // 32
    CH = ROWS_W // KJS
    EW = ROWS_W * IDXW
    NT = N // 16
    mesh = plsc.VectorSubcoreMesh(core_axis_name="c", subcore_axis_name="s")

    @functools.partial(
        pl.kernel,
        out_type=jax.ShapeDtypeStruct((2 * N, 32), F32),
        mesh=mesh,
        compiler_params=pltpu.CompilerParams(use_tc_tiling_on_sc=False),
        scratch_types=[
            pltpu.VMEM_SHARED((N, 32), F32),
            pltpu.VMEM((KJS, IDXW), jnp.int32),
            pltpu.VMEM((KJS * IDXW, 32), F32),
            pltpu.VMEM((IDXW, 32), F32),
        ],
    )
    def scatter_k(v_hbm, dst2_hbm, acc_hbm, shared, idx_v, rows_v, zbuf):
        c = lax.axis_index("c")
        s = lax.axis_index("s")
        wid = s * 2 + c

        z16 = jnp.zeros((16,), F32)

        def zrow(r, carry):
            zbuf[r, pl.ds(0, 16)] = z16
            zbuf[r, pl.ds(16, 16)] = z16
            return carry

        lax.fori_loop(0, IDXW, zrow, 0)

        def zcopy(k, carry):
            pltpu.sync_copy(zbuf, shared.at[pl.ds(s * NT + k * IDXW, IDXW)])
            return carry

        lax.fori_loop(0, NT // IDXW, zcopy, 0)
        plsc.subcore_barrier()

        row0 = wid * ROWS_W
        e0 = wid * EW

        def body(i, carry):
            r0 = row0 + i * KJS
            pltpu.sync_copy(dst2_hbm.at[pl.ds(r0, KJS)], idx_v)
            pltpu.sync_copy(v_hbm.at[pl.ds(e0 + i * (KJS * IDXW), KJS * IDXW)],
                            rows_v)
            for j in range(KJS):
                pltpu.sync_copy(rows_v.at[pl.ds(j * IDXW, IDXW)],
                                shared.at[idx_v.at[j]], add=True)
            return carry

        lax.fori_loop(0, CH, body, 0)
        plsc.subcore_barrier()
        pltpu.sync_copy(shared.at[pl.ds(s * NT, NT)],
                        acc_hbm.at[pl.ds(c * N + s * NT, NT)])

    return scatter_k


def kernel(x, edge_attr, emb, Wn, bn, Wed, bed, W1l, b1l, W1r, b1r, W1e, att1,
           bias1, W2l, b2l, W2r, b2r, W2e, att2, bias2, Wf, bf, edge_index,
           batch):
    N = x.shape[0]
    EH = edge_attr.shape[0]
    E = 2 * EH
    Bj = 6400
    NJ = EH // Bj
    Bn = 2000
    NB = N // Bn

    src, dst = edge_index[0], edge_index[1]
    wm = (src != dst).astype(F32)
    m0 = wm[:EH].reshape(NJ, 1, Bj)
    m1 = wm[EH:].reshape(NJ, 1, Bj)
    src2 = src.reshape(E // IDXW, IDXW)
    dst2 = dst.reshape(E // IDXW, IDXW)
    batch3 = batch.reshape(NB, 1, Bn)

    T = emb @ Wn[:-1]
    wnl = Wn[-1][None, :]
    Wf41 = Wed @ W1e
    Wf42 = Wed @ W2e
    cf1 = bed @ W1e
    cf2 = bed @ W2e

    sred = _ea_reduce(edge_attr, m0, m1, NJ, Bj)
    cnt = jnp.maximum(sred[0, 4], 1.0)
    efm1 = ((sred[0, :4] @ Wf41) / cnt + cf1)[None, :]
    efm2 = ((sred[0, :4] @ Wf42) / cnt + cf2)[None, :]

    tab1 = _front(x, T, wnl, bn[None, :], W1l, b1l[None, :], W1r, b1r[None, :],
                  att1[None, :], efm1, N, NB, Bn)

    gather_k = _make_gather(N, E)
    scatter_k = _make_scatter(N, E)

    gl, gr = gather_k(tab1, src2, dst2)
    bsum1 = (b1l + b1r + cf1)[None, :]
    v = _edge_pass(gl.reshape(2, EH, 32), gr.reshape(2, EH, 32), edge_attr,
                   m0, m1, W1l, W1r, Wf41, bsum1, att1[None, :], E, NJ, Bj)
    acc = scatter_k(v.reshape(E, 32), dst2).reshape(2, N, 32)
    tab2 = _comb_tab(acc, tab1, W1l, (b1l + bias1)[None, :], W2l, b2l[None, :],
                     W2r, b2r[None, :], att2[None, :], efm2, N, NB, Bn)

    gl, gr = gather_k(tab2, src2, dst2)
    bsum2 = (b2l + b2r + cf2)[None, :]
    v = _edge_pass(gl.reshape(2, EH, 32), gr.reshape(2, EH, 32), edge_attr,
                   m0, m1, W2l, W2r, Wf42, bsum2, att2[None, :], E, NJ, Bj)
    acc = scatter_k(v.reshape(E, 32), dst2).reshape(2, N, 32)
    h3 = _comb_h(acc, tab2, W2l, (b2l + bias2)[None, :], N, NB, Bn)

    return _pool_final(h3, batch3, Wf, bf[None, :], NB, Bn)
